# baseline (speedup 1.0000x reference)
import functools

import jax
import jax.numpy as jnp
from jax import lax
from jax.experimental import pallas as pl
from jax.experimental.pallas import tpu as pltpu
from jax.experimental.pallas import tpu_sc as plsc

NN = 10000
EE = 320000
DIN = 128
DOUT = 20
PC = 24
CPT = 3
NSPLIT = 8
NGRP = 4
EPG = EE // NGRP
CH = 8000
NCH = EPG // CH
NTILES = 32


def _pre_body(x_ref, WT_ref, asrc_ref, adst_ref, hT_ref, as_ref, ad_ref, ws_ref):
    hT = jax.lax.dot_general(
        WT_ref[...], x_ref[...], (((1,), (1,)), ((), ())),
        preferred_element_type=jnp.float32)
    hT_ref[0:DOUT, :] = hT
    hT_ref[DOUT:DOUT + 1, :] = jnp.ones((1, NN), jnp.float32)
    hT_ref[DOUT + 1:PC, :] = jnp.zeros((PC - DOUT - 1, NN), jnp.float32)
    a_s = jnp.dot(asrc_ref[...], hT, preferred_element_type=jnp.float32)
    a_d = jnp.dot(adst_ref[...], hT, preferred_element_type=jnp.float32)
    as_ref[...] = a_s.reshape(NN)
    ad_ref[...] = a_d.reshape(NN)
    al = a_s + a_d
    ws_ref[...] = jnp.exp(jnp.maximum(al, 0.2 * al))


_pre_call = pl.pallas_call(
    _pre_body,
    out_shape=[
        jax.ShapeDtypeStruct((PC, NN), jnp.float32),
        jax.ShapeDtypeStruct((NN,), jnp.float32),
        jax.ShapeDtypeStruct((NN,), jnp.float32),
        jax.ShapeDtypeStruct((1, NN), jnp.float32),
    ],
)


EPT_A = EE // NTILES


def _edge_body(hT_hbm, asrc_hbm, adst_hbm, ei_hbm, out_hbm, w_hbm,
               hcols, acc, asv, adv, srcv0, srcv1, dstv0, dstv1, wv0, wv1,
               sem0, sem1):
    srcv = (srcv0, srcv1)
    dstv = (dstv0, dstv1)
    wv = (wv0, wv1)
    sems = (sem0, sem1)
    cc = lax.axis_index("c")
    ss = lax.axis_index("s")
    wid = ss * 2 + cc
    g = cc * 2 + ss // NSPLIT
    k = ss % NSPLIT
    pltpu.sync_copy(hT_hbm.at[pl.ds(k * CPT * NN, CPT * NN)], hcols)
    pltpu.sync_copy(hT_hbm.at[pl.ds((DOUT + 1) * NN, CPT * NN)], acc)
    pltpu.sync_copy(asrc_hbm, asv)
    pltpu.sync_copy(adst_hbm, adv)

    ebase_a = (cc * 2 * EPG) + ss * EPT_A
    for off, ln in ((0, CH), (CH, EPT_A - CH)):
        pltpu.sync_copy(ei_hbm.at[pl.ds(ebase_a + off, ln)], srcv0.at[pl.ds(0, ln)])
        pltpu.sync_copy(ei_hbm.at[pl.ds(EE + ebase_a + off, ln)],
                        dstv0.at[pl.ds(0, ln)])

        @plsc.parallel_loop(0, ln, step=16, unroll=4)
        def w_body(i):
            s16 = srcv0[pl.ds(i, 16)]
            d16 = dstv0[pl.ds(i, 16)]
            al = plsc.load_gather(asv, [s16]) + plsc.load_gather(adv, [d16])
            wv0[pl.ds(i, 16)] = jnp.exp(jnp.maximum(al, 0.2 * al))

        pltpu.sync_copy(wv0.at[pl.ds(0, ln)], w_hbm.at[pl.ds(ebase_a + off, ln)])
    plsc.subcore_barrier()

    ebase = g * EPG

    def _start(ci, b):
        pltpu.make_async_copy(
            ei_hbm.at[pl.ds(ebase + ci * CH, CH)], srcv[b], sems[b]).start()
        pltpu.make_async_copy(
            ei_hbm.at[pl.ds(EE + ebase + ci * CH, CH)], dstv[b], sems[b]).start()
        pltpu.make_async_copy(
            w_hbm.at[pl.ds(ebase + ci * CH, CH)], wv[b], sems[b]).start()

    def _wait(b):
        pltpu.make_async_copy(
            ei_hbm.at[pl.ds(ebase, CH)], srcv[b], sems[b]).wait()
        pltpu.make_async_copy(
            ei_hbm.at[pl.ds(EE + ebase, CH)], dstv[b], sems[b]).wait()
        pltpu.make_async_copy(
            w_hbm.at[pl.ds(ebase, CH)], wv[b], sems[b]).wait()

    _start(0, 0)
    _start(1, 1)

    def chunk_pair(cp, _):
        for b in range(2):
            ci = cp * 2 + b
            _wait(b)

            sv, dv, wvb = srcv[b], dstv[b], wv[b]

            @plsc.parallel_loop(0, CH, step=16, unroll=4)
            def vec_body(i):
                s16 = sv[pl.ds(i, 16)]
                d16 = dv[pl.ds(i, 16)]
                w = wvb[pl.ds(i, 16)]
                for r in range(CPT):
                    hv = plsc.load_gather(hcols, [s16 + (r * NN)])
                    plsc.addupdate_scatter(acc, [d16 + (r * NN)], hv * w)

            @pl.when(ci + 2 < NCH)
            def _():
                _start(ci + 2, b)

        return 0

    lax.fori_loop(0, NCH // 2, chunk_pair, 0)
    pltpu.sync_copy(acc, out_hbm.at[pl.ds(wid * CPT * NN, CPT * NN)])


_edge_call = functools.partial(
    pl.kernel,
    out_type=[
        jax.ShapeDtypeStruct((NTILES * CPT * NN,), jnp.float32),
        jax.ShapeDtypeStruct((EE,), jnp.float32),
    ],
    mesh=plsc.VectorSubcoreMesh(core_axis_name="c", subcore_axis_name="s"),
    compiler_params=pltpu.CompilerParams(needs_layout_passes=False),
    scratch_types=[
        pltpu.VMEM((CPT * NN,), jnp.float32),
        pltpu.VMEM((CPT * NN,), jnp.float32),
        pltpu.VMEM((NN,), jnp.float32),
        pltpu.VMEM((NN,), jnp.float32),
        pltpu.VMEM((CH,), jnp.int32),
        pltpu.VMEM((CH,), jnp.int32),
        pltpu.VMEM((CH,), jnp.int32),
        pltpu.VMEM((CH,), jnp.int32),
        pltpu.VMEM((CH,), jnp.float32),
        pltpu.VMEM((CH,), jnp.float32),
        pltpu.SemaphoreType.DMA,
        pltpu.SemaphoreType.DMA,
    ],
)(_edge_body)


def _stack_body(ei_ref, out_ref, ones_ref):
    out_ref[:, 0:EE] = ei_ref[...]
    out_ref[:, EE:EE + NN] = jax.lax.broadcasted_iota(jnp.int32, (2, NN), 1)
    ones_ref[...] = jnp.ones((1, EE + NN), jnp.float32)


_stack_call = pl.pallas_call(
    _stack_body,
    out_shape=[
        jax.ShapeDtypeStruct((2, EE + NN), jnp.int32),
        jax.ShapeDtypeStruct((1, EE + NN), jnp.float32),
    ],
)


def _combine_body(A_ref, hT_ref, ws_ref, bias_ref, out_ref):
    ns = (A_ref[0:PC, :] + A_ref[PC:2 * PC, :]
          + A_ref[2 * PC:3 * PC, :] + A_ref[3 * PC:4 * PC, :])
    ws = ws_ref[...]
    num = ns[0:DOUT, :] + ws * hT_ref[0:DOUT, :]
    den = ns[DOUT:DOUT + 1, :] + ws + 1e-16
    out_ref[...] = num / den + bias_ref[...]


_combine_call = pl.pallas_call(
    _combine_body,
    out_shape=jax.ShapeDtypeStruct((DOUT, NN), jnp.float32),
)


def kernel(x, edge_index, W, att_src, att_dst, bias):
    hT, a_s, a_d, ws = _pre_call(
        x, W.T, att_src.reshape(1, DOUT), att_dst.reshape(1, DOUT))
    A, _unused_w = _edge_call(hT.reshape(-1), a_s, a_d, edge_index.reshape(-1))
    outT = _combine_call(A.reshape(NGRP * PC, NN), hT, ws, bias.reshape(DOUT, 1))
    out = outT.T
    stacked, ones_row = _stack_call(edge_index)
    attn = ones_row.reshape(EE + NN, 1)
    return out, (stacked, attn)


# speedup vs baseline: 1.0184x; 1.0184x over previous
import functools

import jax
import jax.numpy as jnp
from jax import lax
from jax.experimental import pallas as pl
from jax.experimental.pallas import tpu as pltpu
from jax.experimental.pallas import tpu_sc as plsc

NN = 10000
EE = 320000
DIN = 128
DOUT = 20
PC = 24
CPT = 3
NSPLIT = 8
NGRP = 4
EPG = EE // NGRP
CH = 10000
NCH = EPG // CH
NTILES = 32


def _pre_body(x_ref, WT_ref, asrc_ref, adst_ref, hT_ref, as_ref, ad_ref, ws_ref):
    hT = jax.lax.dot_general(
        WT_ref[...], x_ref[...], (((1,), (1,)), ((), ())),
        preferred_element_type=jnp.float32)
    hT_ref[0:DOUT, :] = hT
    hT_ref[DOUT:DOUT + 1, :] = jnp.ones((1, NN), jnp.float32)
    hT_ref[DOUT + 1:PC, :] = jnp.zeros((PC - DOUT - 1, NN), jnp.float32)
    a_s = jnp.dot(asrc_ref[...], hT, preferred_element_type=jnp.float32)
    a_d = jnp.dot(adst_ref[...], hT, preferred_element_type=jnp.float32)
    as_ref[...] = a_s.reshape(NN)
    ad_ref[...] = a_d.reshape(NN)
    al = a_s + a_d
    ws_ref[...] = jnp.exp(jnp.maximum(al, 0.2 * al))


_pre_call = pl.pallas_call(
    _pre_body,
    out_shape=[
        jax.ShapeDtypeStruct((PC, NN), jnp.float32),
        jax.ShapeDtypeStruct((NN,), jnp.float32),
        jax.ShapeDtypeStruct((NN,), jnp.float32),
        jax.ShapeDtypeStruct((1, NN), jnp.float32),
    ],
)


def _edge_body(hT_hbm, asrc_hbm, adst_hbm, ei_hbm, out_hbm,
               hcols, acc, asv, adv, srcv0, srcv1, dstv0, dstv1, sem0, sem1):
    srcv = (srcv0, srcv1)
    dstv = (dstv0, dstv1)
    sems = (sem0, sem1)
    wid = lax.axis_index("s") * 2 + lax.axis_index("c")
    g = wid // NSPLIT
    k = wid % NSPLIT
    pltpu.sync_copy(hT_hbm.at[pl.ds(k * CPT * NN, CPT * NN)], hcols)
    pltpu.sync_copy(hT_hbm.at[pl.ds((DOUT + 1) * NN, CPT * NN)], acc)
    pltpu.sync_copy(asrc_hbm, asv)
    pltpu.sync_copy(adst_hbm, adv)
    ebase = g * EPG

    def _start(ci, b):
        pltpu.make_async_copy(
            ei_hbm.at[pl.ds(ebase + ci * CH, CH)], srcv[b], sems[b]).start()
        pltpu.make_async_copy(
            ei_hbm.at[pl.ds(EE + ebase + ci * CH, CH)], dstv[b], sems[b]).start()

    def _wait(b):
        pltpu.make_async_copy(
            ei_hbm.at[pl.ds(ebase, CH)], srcv[b], sems[b]).wait()
        pltpu.make_async_copy(
            ei_hbm.at[pl.ds(EE + ebase, CH)], dstv[b], sems[b]).wait()

    _start(0, 0)
    _start(1, 1)

    def chunk_pair(cp, _):
        for b in range(2):
            ci = cp * 2 + b
            _wait(b)

            sv, dv = srcv[b], dstv[b]

            @plsc.parallel_loop(0, CH, step=16, unroll=4)
            def vec_body(i):
                s16 = sv[pl.ds(i, 16)]
                d16 = dv[pl.ds(i, 16)]
                a1 = plsc.load_gather(asv, [s16])
                a2 = plsc.load_gather(adv, [d16])
                al = a1 + a2
                w = jnp.exp(jnp.maximum(al, 0.2 * al))
                for r in range(CPT):
                    hv = plsc.load_gather(hcols, [s16 + (r * NN)])
                    plsc.addupdate_scatter(acc, [d16 + (r * NN)], hv * w)

            @pl.when(ci + 2 < NCH)
            def _():
                _start(ci + 2, b)

        return 0

    lax.fori_loop(0, NCH // 2, chunk_pair, 0)
    pltpu.sync_copy(acc, out_hbm.at[pl.ds(wid * CPT * NN, CPT * NN)])


_edge_call = functools.partial(
    pl.kernel,
    out_type=jax.ShapeDtypeStruct((NTILES * CPT * NN,), jnp.float32),
    mesh=plsc.VectorSubcoreMesh(core_axis_name="c", subcore_axis_name="s"),
    compiler_params=pltpu.CompilerParams(needs_layout_passes=False),
    scratch_types=[
        pltpu.VMEM((CPT * NN,), jnp.float32),
        pltpu.VMEM((CPT * NN,), jnp.float32),
        pltpu.VMEM((NN,), jnp.float32),
        pltpu.VMEM((NN,), jnp.float32),
        pltpu.VMEM((CH,), jnp.int32),
        pltpu.VMEM((CH,), jnp.int32),
        pltpu.VMEM((CH,), jnp.int32),
        pltpu.VMEM((CH,), jnp.int32),
        pltpu.SemaphoreType.DMA,
        pltpu.SemaphoreType.DMA,
    ],
)(_edge_body)


def _stack_body(ei_ref, out_ref, ones_ref):
    out_ref[:, 0:EE] = ei_ref[...]
    out_ref[:, EE:EE + NN] = jax.lax.broadcasted_iota(jnp.int32, (2, NN), 1)
    ones_ref[...] = jnp.ones((1, EE + NN), jnp.float32)


_stack_call = pl.pallas_call(
    _stack_body,
    out_shape=[
        jax.ShapeDtypeStruct((2, EE + NN), jnp.int32),
        jax.ShapeDtypeStruct((1, EE + NN), jnp.float32),
    ],
)


def _combine_body(A_ref, hT_ref, ws_ref, bias_ref, out_ref):
    ns = (A_ref[0:PC, :] + A_ref[PC:2 * PC, :]
          + A_ref[2 * PC:3 * PC, :] + A_ref[3 * PC:4 * PC, :])
    ws = ws_ref[...]
    num = ns[0:DOUT, :] + ws * hT_ref[0:DOUT, :]
    den = ns[DOUT:DOUT + 1, :] + ws + 1e-16
    out_ref[...] = num / den + bias_ref[...]


_combine_call = pl.pallas_call(
    _combine_body,
    out_shape=jax.ShapeDtypeStruct((DOUT, NN), jnp.float32),
)


def kernel(x, edge_index, W, att_src, att_dst, bias):
    hT, a_s, a_d, ws = _pre_call(
        x, W.T, att_src.reshape(1, DOUT), att_dst.reshape(1, DOUT))
    A = _edge_call(hT.reshape(-1), a_s, a_d, edge_index.reshape(-1))
    outT = _combine_call(A.reshape(NGRP * PC, NN), hT, ws, bias.reshape(DOUT, 1))
    out = outT.T
    stacked, ones_row = _stack_call(edge_index)
    attn = ones_row.reshape(EE + NN, 1)
    return out, (stacked, attn)


# speedup vs baseline: 1.0257x; 1.0072x over previous
import functools

import jax
import jax.numpy as jnp
from jax import lax
from jax.experimental import pallas as pl
from jax.experimental.pallas import tpu as pltpu
from jax.experimental.pallas import tpu_sc as plsc

NN = 10000
EE = 320000
DIN = 128
DOUT = 20
PC = 24
CPT = 3
NSPLIT = 8
NGRP = 4
EPG = EE // NGRP
CH = 8000
NCH = EPG // CH
NTILES = 32


def _pre_body(x_ref, WT_ref, asrc_ref, adst_ref, hT_ref, as_ref, ad_ref, ws_ref):
    hT = jax.lax.dot_general(
        WT_ref[...], x_ref[...], (((1,), (1,)), ((), ())),
        preferred_element_type=jnp.float32)
    hT_ref[0:DOUT, :] = hT
    hT_ref[DOUT:DOUT + 1, :] = jnp.ones((1, NN), jnp.float32)
    hT_ref[DOUT + 1:PC, :] = jnp.zeros((PC - DOUT - 1, NN), jnp.float32)
    a_s = jnp.dot(asrc_ref[...], hT, preferred_element_type=jnp.float32)
    a_d = jnp.dot(adst_ref[...], hT, preferred_element_type=jnp.float32)
    as_ref[...] = a_s.reshape(NN)
    ad_ref[...] = a_d.reshape(NN)
    al = a_s + a_d
    ws_ref[...] = jnp.exp(jnp.maximum(al, 0.2 * al))


_pre_call = pl.pallas_call(
    _pre_body,
    out_shape=[
        jax.ShapeDtypeStruct((PC, NN), jnp.float32),
        jax.ShapeDtypeStruct((NN,), jnp.float32),
        jax.ShapeDtypeStruct((NN,), jnp.float32),
        jax.ShapeDtypeStruct((1, NN), jnp.float32),
    ],
)


def _edge_body(hT_hbm, asrc_hbm, adst_hbm, ei_hbm, out_hbm,
               hcols, acc, asv, adv, srcv0, srcv1, dstv0, dstv1, sem0, sem1):
    srcv = (srcv0, srcv1)
    dstv = (dstv0, dstv1)
    sems = (sem0, sem1)
    wid = lax.axis_index("s") * 2 + lax.axis_index("c")
    g = wid // NSPLIT
    k = wid % NSPLIT
    pltpu.sync_copy(hT_hbm.at[pl.ds(k * CPT * NN, CPT * NN)], hcols)
    pltpu.sync_copy(hT_hbm.at[pl.ds((DOUT + 1) * NN, CPT * NN)], acc)
    pltpu.sync_copy(asrc_hbm, asv)
    pltpu.sync_copy(adst_hbm, adv)
    ebase = g * EPG

    def _start(ci, b):
        pltpu.make_async_copy(
            ei_hbm.at[pl.ds(ebase + ci * CH, CH)], srcv[b], sems[b]).start()
        pltpu.make_async_copy(
            ei_hbm.at[pl.ds(EE + ebase + ci * CH, CH)], dstv[b], sems[b]).start()

    def _wait(b):
        pltpu.make_async_copy(
            ei_hbm.at[pl.ds(ebase, CH)], srcv[b], sems[b]).wait()
        pltpu.make_async_copy(
            ei_hbm.at[pl.ds(EE + ebase, CH)], dstv[b], sems[b]).wait()

    _start(0, 0)
    _start(1, 1)

    def chunk_pair(cp, _):
        for b in range(2):
            ci = cp * 2 + b
            _wait(b)

            sv, dv = srcv[b], dstv[b]

            @plsc.parallel_loop(0, CH, step=16, unroll=4)
            def vec_body(i):
                s16 = sv[pl.ds(i, 16)]
                d16 = dv[pl.ds(i, 16)]
                a1 = plsc.load_gather(asv, [s16])
                a2 = plsc.load_gather(adv, [d16])
                al = a1 + a2
                w = jnp.exp(jnp.maximum(al, 0.2 * al))
                for r in range(CPT):
                    hv = plsc.load_gather(hcols, [s16 + (r * NN)])
                    plsc.addupdate_scatter(acc, [d16 + (r * NN)], hv * w)

            @pl.when(ci + 2 < NCH)
            def _():
                _start(ci + 2, b)

        return 0

    lax.fori_loop(0, NCH // 2, chunk_pair, 0)
    pltpu.sync_copy(acc, out_hbm.at[pl.ds(wid * CPT * NN, CPT * NN)])


_edge_call = functools.partial(
    pl.kernel,
    out_type=jax.ShapeDtypeStruct((NTILES * CPT * NN,), jnp.float32),
    mesh=plsc.VectorSubcoreMesh(core_axis_name="c", subcore_axis_name="s"),
    compiler_params=pltpu.CompilerParams(needs_layout_passes=False),
    scratch_types=[
        pltpu.VMEM((CPT * NN,), jnp.float32),
        pltpu.VMEM((CPT * NN,), jnp.float32),
        pltpu.VMEM((NN,), jnp.float32),
        pltpu.VMEM((NN,), jnp.float32),
        pltpu.VMEM((CH,), jnp.int32),
        pltpu.VMEM((CH,), jnp.int32),
        pltpu.VMEM((CH,), jnp.int32),
        pltpu.VMEM((CH,), jnp.int32),
        pltpu.SemaphoreType.DMA,
        pltpu.SemaphoreType.DMA,
    ],
)(_edge_body)


def _stack_body(ei_ref, out_ref, ones_ref):
    out_ref[:, 0:EE] = ei_ref[...]
    out_ref[:, EE:EE + NN] = jax.lax.broadcasted_iota(jnp.int32, (2, NN), 1)
    ones_ref[...] = jnp.ones((1, EE + NN), jnp.float32)


_stack_call = pl.pallas_call(
    _stack_body,
    out_shape=[
        jax.ShapeDtypeStruct((2, EE + NN), jnp.int32),
        jax.ShapeDtypeStruct((1, EE + NN), jnp.float32),
    ],
)


def _combine_body(A_ref, hT_ref, ws_ref, bias_ref, out_ref):
    ns = (A_ref[0:PC, :] + A_ref[PC:2 * PC, :]
          + A_ref[2 * PC:3 * PC, :] + A_ref[3 * PC:4 * PC, :])
    ws = ws_ref[...]
    num = ns[0:DOUT, :] + ws * hT_ref[0:DOUT, :]
    den = ns[DOUT:DOUT + 1, :] + ws + 1e-16
    out_ref[...] = num / den + bias_ref[...]


_combine_call = pl.pallas_call(
    _combine_body,
    out_shape=jax.ShapeDtypeStruct((DOUT, NN), jnp.float32),
)


def kernel(x, edge_index, W, att_src, att_dst, bias):
    hT, a_s, a_d, ws = _pre_call(
        x, W.T, att_src.reshape(1, DOUT), att_dst.reshape(1, DOUT))
    A = _edge_call(hT.reshape(-1), a_s, a_d, edge_index.reshape(-1))
    outT = _combine_call(A.reshape(NGRP * PC, NN), hT, ws, bias.reshape(DOUT, 1))
    out = outT.T
    stacked, ones_row = _stack_call(edge_index)
    attn = ones_row.reshape(EE + NN, 1)
    return out, (stacked, attn)
